# baseline (speedup 1.0000x reference)
import functools

import jax
import jax.numpy as jnp
from jax import lax
from jax.experimental import pallas as pl
from jax.experimental.pallas import tpu as pltpu
from jax.experimental.pallas import tpu_sc as plsc

N = 10000
E = 320000
D = 128
NUM_ITERS = 4
GAMMA = 0.1
EPS = 0.1

NC = 2
NS = 16
NW = NC * NS

C = 128
NBUF = 2
PH = 2
CH = NBUF * PH * (-(-E // (NW * C * NBUF * PH)))
CHP = CH // PH
EPT = CH * C
E_PAD = EPT * NW

RPT = 632
N_PAD = RPT * NS

_mesh = plsc.VectorSubcoreMesh(
    core_axis_name="c", subcore_axis_name="s", num_cores=NC, num_subcores=NS
)


@functools.partial(
    pl.kernel,
    out_type=jax.ShapeDtypeStruct((NC, N_PAD, D), jnp.float32),
    mesh=_mesh,
    scratch_types=[
        pltpu.VMEM((CHP, C), jnp.int32),
        pltpu.VMEM((CHP, C), jnp.int32),
        [pltpu.VMEM((C, D), jnp.float32) for _ in range(NBUF)],
        pltpu.VMEM_SHARED((N_PAD, D), jnp.float32),
        [pltpu.SemaphoreType.DMA for _ in range(NBUF)],
    ],
)
def _sc_propagate(h_hbm, src_hbm, dst_hbm, out_hbm, sidx, didx, rows, acc, sems):
    cid = lax.axis_index("c")
    sid = lax.axis_index("s")
    wid = sid * NC + cid

    def _zero(i, _):
        rows[0][i // 8, pl.ds((i % 8) * 16, 16)] = jnp.zeros((16,), jnp.float32)
        return 0

    lax.fori_loop(0, C * (D // 16), _zero, 0)

    base = sid * RPT
    spans = [(0, C), (C, C), (2 * C, C), (3 * C, C), (4 * C, RPT - 4 * C)]
    for off, sz in spans:
        pltpu.sync_copy(rows[0].at[pl.ds(0, sz)], acc.at[pl.ds(base + off, sz)])
    plsc.subcore_barrier()

    for ph in range(PH):
        pltpu.sync_copy(src_hbm.at[wid, pl.ds(ph * CHP, CHP)], sidx)
        pltpu.sync_copy(dst_hbm.at[wid, pl.ds(ph * CHP, CHP)], didx)
        for b in range(NBUF):
            pltpu.async_copy(h_hbm.at[sidx.at[b]], rows[b], sems[b])

        def _round(j2, fire_next):
            for b in range(NBUF):
                c = j2 * NBUF + b
                pltpu.make_async_copy(h_hbm.at[sidx.at[b]], rows[b], sems[b]).wait()
                pltpu.sync_copy(rows[b], acc.at[didx.at[c]], add=True)
                if fire_next:
                    pltpu.async_copy(h_hbm.at[sidx.at[c + NBUF]], rows[b], sems[b])

        def _body(j2, _):
            _round(j2, True)
            return 0

        lax.fori_loop(0, CHP // NBUF - 1, _body, 0)
        _round(CHP // NBUF - 1, False)
    plsc.subcore_barrier()

    for off, sz in spans:
        pltpu.sync_copy(acc.at[pl.ds(base + off, sz)], rows[0].at[pl.ds(0, sz)])
        pltpu.sync_copy(rows[0].at[pl.ds(0, sz)], out_hbm.at[cid, pl.ds(base + off, sz)])


def _embed_body(x_ref, w_ref, b_ref, o_ref):
    z = lax.dot_general(
        x_ref[...], w_ref[...], (((1,), (1,)), ((), ())),
        preferred_element_type=jnp.float32,
    )
    o_ref[...] = jnp.maximum(z + b_ref[...], 0.0)


def _update_body(h_ref, p0_ref, p1_ref, w_ref, lw_ref, b_ref, o_ref):
    h = h_ref[...]
    p = p0_ref[...] + p1_ref[...]
    hwt = lax.dot_general(h, w_ref[...], (((1,), (1,)), ((), ())),
                          preferred_element_type=jnp.float32)
    hw = lax.dot_general(h, w_ref[...], (((1,), (0,)), ((), ())),
                         preferred_element_type=jnp.float32)
    plw = lax.dot_general(p, lw_ref[...], (((1,), (1,)), ((), ())),
                          preferred_element_type=jnp.float32)
    conv = hwt - hw - GAMMA * h + plw + b_ref[...]
    o_ref[...] = h + EPS * jnp.tanh(conv)


def _readout_body(h_ref, w_ref, b_ref, o_ref):
    z = lax.dot_general(
        h_ref[...], w_ref[...], (((1,), (1,)), ((), ())),
        preferred_element_type=jnp.float32,
    )
    o_ref[...] = z + b_ref[...]


_ROWS_B = 1000
_GRID = (N // _ROWS_B,)
_row_spec = pl.BlockSpec((_ROWS_B, D), lambda i: (i, 0))
_mat_spec = pl.BlockSpec((D, D), lambda i: (0, 0))
_vec_spec = pl.BlockSpec((1, D), lambda i: (0, 0))
_out_struct = jax.ShapeDtypeStruct((N, D), jnp.float32)


def _tc_embed(x, w, b2):
    return pl.pallas_call(
        _embed_body, grid=_GRID,
        in_specs=[_row_spec, _mat_spec, _vec_spec],
        out_specs=_row_spec, out_shape=_out_struct,
    )(x, w, b2)


def _tc_update(h, p0, p1, w, lw, b2):
    return pl.pallas_call(
        _update_body, grid=_GRID,
        in_specs=[_row_spec, _row_spec, _row_spec, _mat_spec, _mat_spec, _vec_spec],
        out_specs=_row_spec, out_shape=_out_struct,
    )(h, p0, p1, w, lw, b2)


def _tc_readout(h, w, b2):
    return pl.pallas_call(
        _readout_body, grid=_GRID,
        in_specs=[_row_spec, _mat_spec, _vec_spec],
        out_specs=_row_spec, out_shape=_out_struct,
    )(h, w, b2)


def kernel(x, edge_index, emb_W, emb_b, W, bias, lin_W, readout_W, readout_b):
    ei = edge_index.astype(jnp.int32)
    pad = E_PAD - E
    pad_dst = N + (jnp.arange(pad, dtype=jnp.int32) % (N_PAD - N))
    src = jnp.concatenate([ei[0], jnp.zeros((pad,), jnp.int32)]).reshape(NW, CH, C)
    dst = jnp.concatenate([ei[1], pad_dst]).reshape(NW, CH, C)

    emb_b2 = emb_b.reshape(1, D)
    bias2 = bias.reshape(1, D)
    ro_b2 = readout_b.reshape(1, D)

    h = _tc_embed(x, emb_W, emb_b2)
    for _ in range(NUM_ITERS):
        parts = _sc_propagate(h, src, dst)
        p0 = parts[0, :N, :]
        p1 = parts[1, :N, :]
        h = _tc_update(h, p0, p1, W, lin_W, bias2)
    return _tc_readout(h, readout_W, ro_b2)


# speedup vs baseline: 3.5058x; 3.5058x over previous
import functools

import jax
import jax.numpy as jnp
from jax import lax
from jax.experimental import pallas as pl
from jax.experimental.pallas import tpu as pltpu
from jax.experimental.pallas import tpu_sc as plsc

N = 10000
E = 320000
D = 128
NUM_ITERS = 4
GAMMA = 0.1
EPS = 0.1

NC = 2
NS = 16
NW = NC * NS

C = 128
NBUF = 2
PH = 2
CH = NBUF * PH * (-(-E // (NW * C * NBUF * PH)))
CHP = CH // PH
EPT = CH * C
E_PAD = EPT * NW

RPT = 632
N_PAD = RPT * NS

_mesh = plsc.VectorSubcoreMesh(
    core_axis_name="c", subcore_axis_name="s", num_cores=NC, num_subcores=NS
)


@functools.partial(
    pl.kernel,
    out_type=jax.ShapeDtypeStruct((NC, N_PAD, D), jnp.float32),
    mesh=_mesh,
    scratch_types=[
        pltpu.VMEM((CHP, C), jnp.int32),
        pltpu.VMEM((CHP, C), jnp.int32),
        [pltpu.VMEM((C, D), jnp.float32) for _ in range(NBUF)],
        pltpu.VMEM_SHARED((N_PAD, D), jnp.float32),
        [pltpu.SemaphoreType.DMA for _ in range(NBUF)],
    ],
)
def _sc_propagate(h_hbm, src_hbm, dst_hbm, out_hbm, sidx, didx, rows, acc, sems):
    cid = lax.axis_index("c")
    sid = lax.axis_index("s")
    wid = sid * NC + cid

    def _zero(i, _):
        rows[0][i // 8, pl.ds((i % 8) * 16, 16)] = jnp.zeros((16,), jnp.float32)
        return 0

    lax.fori_loop(0, C * (D // 16), _zero, 0)

    base = sid * RPT
    spans = [(0, C), (C, C), (2 * C, C), (3 * C, C), (4 * C, RPT - 4 * C)]
    for off, sz in spans:
        pltpu.sync_copy(rows[0].at[pl.ds(0, sz)], acc.at[pl.ds(base + off, sz)])
    plsc.subcore_barrier()

    for ph in range(PH):
        pltpu.sync_copy(src_hbm.at[wid, pl.ds(ph * CHP, CHP)], sidx)
        pltpu.sync_copy(dst_hbm.at[wid, pl.ds(ph * CHP, CHP)], didx)
        for b in range(NBUF):
            pltpu.async_copy(h_hbm.at[sidx.at[b]], rows[b], sems[b])

        def _round(j2, fire_next):
            for b in range(NBUF):
                c = j2 * NBUF + b
                pltpu.make_async_copy(h_hbm.at[sidx.at[b]], rows[b], sems[b]).wait()
                pltpu.sync_copy(rows[b], acc.at[didx.at[c]], add=True)
                if fire_next:
                    pltpu.async_copy(h_hbm.at[sidx.at[c + NBUF]], rows[b], sems[b])

        def _body(j2, _):
            _round(j2, True)
            return 0

        lax.fori_loop(0, CHP // NBUF - 1, _body, 0)
        _round(CHP // NBUF - 1, False)
    plsc.subcore_barrier()

    for off, sz in spans:
        pltpu.sync_copy(acc.at[pl.ds(base + off, sz)], rows[0].at[pl.ds(0, sz)])
        pltpu.sync_copy(rows[0].at[pl.ds(0, sz)], out_hbm.at[cid, pl.ds(base + off, sz)])


def _embed_body(x_ref, w_ref, b_ref, o_ref):
    z = lax.dot_general(
        x_ref[...], w_ref[...], (((1,), (1,)), ((), ())),
        preferred_element_type=jnp.float32,
    )
    o_ref[...] = jnp.maximum(z + b_ref[...], 0.0)


def _update_body(h_ref, p0_ref, p1_ref, w_ref, lw_ref, b_ref, o_ref):
    h = h_ref[...]
    p = p0_ref[...] + p1_ref[...]
    hwt = lax.dot_general(h, w_ref[...], (((1,), (1,)), ((), ())),
                          preferred_element_type=jnp.float32)
    hw = lax.dot_general(h, w_ref[...], (((1,), (0,)), ((), ())),
                         preferred_element_type=jnp.float32)
    plw = lax.dot_general(p, lw_ref[...], (((1,), (1,)), ((), ())),
                          preferred_element_type=jnp.float32)
    conv = hwt - hw - GAMMA * h + plw + b_ref[...]
    o_ref[...] = h + EPS * jnp.tanh(conv)


def _readout_body(h_ref, w_ref, b_ref, o_ref):
    z = lax.dot_general(
        h_ref[...], w_ref[...], (((1,), (1,)), ((), ())),
        preferred_element_type=jnp.float32,
    )
    o_ref[...] = z + b_ref[...]


_ROWS_B = 1000
_GRID = (N // _ROWS_B,)
_row_spec = pl.BlockSpec((_ROWS_B, D), lambda i: (i, 0))
_mat_spec = pl.BlockSpec((D, D), lambda i: (0, 0))
_vec_spec = pl.BlockSpec((1, D), lambda i: (0, 0))
_out_struct = jax.ShapeDtypeStruct((N, D), jnp.float32)


def _tc_embed(x, w, b2):
    return pl.pallas_call(
        _embed_body, grid=_GRID,
        in_specs=[_row_spec, _mat_spec, _vec_spec],
        out_specs=_row_spec, out_shape=_out_struct,
    )(x, w, b2)


def _tc_update(h, p0, p1, w, lw, b2):
    return pl.pallas_call(
        _update_body, grid=_GRID,
        in_specs=[_row_spec, _row_spec, _row_spec, _mat_spec, _mat_spec, _vec_spec],
        out_specs=_row_spec, out_shape=_out_struct,
    )(h, p0, p1, w, lw, b2)


def _tc_readout(h, w, b2):
    return pl.pallas_call(
        _readout_body, grid=_GRID,
        in_specs=[_row_spec, _mat_spec, _vec_spec],
        out_specs=_row_spec, out_shape=_out_struct,
    )(h, w, b2)


def kernel(x, edge_index, emb_W, emb_b, W, bias, lin_W, readout_W, readout_b):
    ei = edge_index.astype(jnp.int32)
    pad = E_PAD - E
    pad_ar = jnp.arange(pad, dtype=jnp.int32)
    pad_src = pad_ar % N
    pad_dst = N + (pad_ar % (N_PAD - N))
    src = jnp.concatenate([ei[0], pad_src]).reshape(NW, CH, C)
    dst = jnp.concatenate([ei[1], pad_dst]).reshape(NW, CH, C)

    emb_b2 = emb_b.reshape(1, D)
    bias2 = bias.reshape(1, D)
    ro_b2 = readout_b.reshape(1, D)

    h = _tc_embed(x, emb_W, emb_b2)
    for _ in range(NUM_ITERS):
        parts = _sc_propagate(h, src, dst)
        p0 = parts[0, :N, :]
        p1 = parts[1, :N, :]
        h = _tc_update(h, p0, p1, W, lin_W, bias2)
    return _tc_readout(h, readout_W, ro_b2)


# speedup vs baseline: 3.7365x; 1.0658x over previous
import functools

import numpy as np

import jax
import jax.numpy as jnp
from jax import lax
from jax.experimental import pallas as pl
from jax.experimental.pallas import tpu as pltpu
from jax.experimental.pallas import tpu_sc as plsc

N = 10000
E = 320000
D = 128
NUM_ITERS = 4
GAMMA = 0.1
EPS = 0.1

NC = 2
NS = 16
NW = NC * NS

C = 128
NBUF = 2
PH = 2
CH = NBUF * PH * (-(-E // (NW * C * NBUF * PH)))
CHP = CH // PH
EPT = CH * C
E_PAD = EPT * NW

RPT = 632
N_PAD = RPT * NS

_mesh = plsc.VectorSubcoreMesh(
    core_axis_name="c", subcore_axis_name="s", num_cores=NC, num_subcores=NS
)


@functools.partial(
    pl.kernel,
    out_type=jax.ShapeDtypeStruct((NC, N_PAD, D), jnp.float32),
    mesh=_mesh,
    scratch_types=[
        pltpu.VMEM((CHP, C), jnp.int32),
        pltpu.VMEM((CHP, C), jnp.int32),
        [pltpu.VMEM((C, D), jnp.float32) for _ in range(NBUF)],
        pltpu.VMEM_SHARED((N_PAD, D), jnp.float32),
        [pltpu.SemaphoreType.DMA for _ in range(NBUF)],
    ],
)
def _sc_propagate(h_hbm, src_hbm, dst_hbm, out_hbm, sidx, didx, rows, acc, sems):
    cid = lax.axis_index("c")
    sid = lax.axis_index("s")
    wid = sid * NC + cid

    def _zero(i, _):
        rows[0][i // 8, pl.ds((i % 8) * 16, 16)] = jnp.zeros((16,), jnp.float32)
        return 0

    lax.fori_loop(0, C * (D // 16), _zero, 0)

    base = sid * RPT
    spans = [(0, C), (C, C), (2 * C, C), (3 * C, C), (4 * C, RPT - 4 * C)]
    for off, sz in spans:
        pltpu.sync_copy(rows[0].at[pl.ds(0, sz)], acc.at[pl.ds(base + off, sz)])
    plsc.subcore_barrier()

    for ph in range(PH):
        pltpu.sync_copy(src_hbm.at[wid, pl.ds(ph * CHP, CHP)], sidx)
        pltpu.sync_copy(dst_hbm.at[wid, pl.ds(ph * CHP, CHP)], didx)
        for b in range(NBUF):
            pltpu.async_copy(h_hbm.at[sidx.at[b]], rows[b], sems[b])

        def _round(j2, fire_next):
            for b in range(NBUF):
                c = j2 * NBUF + b
                pltpu.make_async_copy(h_hbm.at[sidx.at[b]], rows[b], sems[b]).wait()
                pltpu.sync_copy(rows[b], acc.at[didx.at[c]], add=True)
                if fire_next:
                    pltpu.async_copy(h_hbm.at[sidx.at[c + NBUF]], rows[b], sems[b])

        def _body(j2, _):
            _round(j2, True)
            return 0

        lax.fori_loop(0, CHP // NBUF - 1, _body, 0)
        _round(CHP // NBUF - 1, False)
    plsc.subcore_barrier()

    pltpu.sync_copy(acc.at[pl.ds(base, RPT)], out_hbm.at[cid, pl.ds(base, RPT)])


def _embed_body(x_ref, w_ref, b_ref, o_ref):
    z = lax.dot_general(
        x_ref[...], w_ref[...], (((1,), (1,)), ((), ())),
        preferred_element_type=jnp.float32,
    )
    o_ref[...] = jnp.maximum(z + b_ref[...], 0.0)


def _new_h(h_ref, p_ref, w_ref, lw_ref, b_ref):
    h = h_ref[...]
    p = p_ref[0] + p_ref[1]
    hwt = lax.dot_general(h, w_ref[...], (((1,), (1,)), ((), ())),
                          preferred_element_type=jnp.float32)
    hw = lax.dot_general(h, w_ref[...], (((1,), (0,)), ((), ())),
                         preferred_element_type=jnp.float32)
    plw = lax.dot_general(p, lw_ref[...], (((1,), (1,)), ((), ())),
                          preferred_element_type=jnp.float32)
    conv = hwt - hw - GAMMA * h + plw + b_ref[...]
    return h + EPS * jnp.tanh(conv)


def _update_body(h_ref, p_ref, w_ref, lw_ref, b_ref, o_ref):
    o_ref[...] = _new_h(h_ref, p_ref, w_ref, lw_ref, b_ref)


def _update_readout_body(h_ref, p_ref, w_ref, lw_ref, b_ref, rw_ref, rb_ref, o_ref):
    hn = _new_h(h_ref, p_ref, w_ref, lw_ref, b_ref)
    z = lax.dot_general(hn, rw_ref[...], (((1,), (1,)), ((), ())),
                        preferred_element_type=jnp.float32)
    o_ref[...] = z + rb_ref[...]


_ROWS_B = 1000
_GRID = (N // _ROWS_B,)
_row_spec = pl.BlockSpec((_ROWS_B, D), lambda i: (i, 0))
_parts_spec = pl.BlockSpec((NC, _ROWS_B, D), lambda i: (0, i, 0))
_mat_spec = pl.BlockSpec((D, D), lambda i: (0, 0))
_vec_spec = pl.BlockSpec((1, D), lambda i: (0, 0))
_out_struct = jax.ShapeDtypeStruct((N, D), jnp.float32)


def _tc_embed(x, w, b2):
    return pl.pallas_call(
        _embed_body, grid=_GRID,
        in_specs=[_row_spec, _mat_spec, _vec_spec],
        out_specs=_row_spec, out_shape=_out_struct,
    )(x, w, b2)


def _tc_update(h, parts, w, lw, b2):
    return pl.pallas_call(
        _update_body, grid=_GRID,
        in_specs=[_row_spec, _parts_spec, _mat_spec, _mat_spec, _vec_spec],
        out_specs=_row_spec, out_shape=_out_struct,
    )(h, parts, w, lw, b2)


def _tc_update_readout(h, parts, w, lw, b2, rw, rb2):
    return pl.pallas_call(
        _update_readout_body, grid=_GRID,
        in_specs=[_row_spec, _parts_spec, _mat_spec, _mat_spec, _vec_spec,
                  _mat_spec, _vec_spec],
        out_specs=_row_spec, out_shape=_out_struct,
    )(h, parts, w, lw, b2, rw, rb2)


def kernel(x, edge_index, emb_W, emb_b, W, bias, lin_W, readout_W, readout_b):
    ei = edge_index.astype(jnp.int32)
    pad = E_PAD - E
    pad_ar = np.arange(pad, dtype=np.int32)
    pad_src = jnp.asarray(pad_ar % N)
    pad_dst = jnp.asarray(N + (pad_ar % (N_PAD - N)))
    src = jnp.concatenate([ei[0], pad_src]).reshape(NW, CH, C)
    dst = jnp.concatenate([ei[1], pad_dst]).reshape(NW, CH, C)

    emb_b2 = emb_b.reshape(1, D)
    bias2 = bias.reshape(1, D)
    ro_b2 = readout_b.reshape(1, D)

    h = _tc_embed(x, emb_W, emb_b2)
    for _ in range(NUM_ITERS - 1):
        parts = _sc_propagate(h, src, dst)
        h = _tc_update(h, parts, W, lin_W, bias2)
    parts = _sc_propagate(h, src, dst)
    return _tc_update_readout(h, parts, W, lin_W, bias2, readout_W, ro_b2)


# speedup vs baseline: 3.9529x; 1.0579x over previous
import functools

import numpy as np

import jax
import jax.numpy as jnp
from jax import lax
from jax.experimental import pallas as pl
from jax.experimental.pallas import tpu as pltpu
from jax.experimental.pallas import tpu_sc as plsc

N = 10000
E = 320000
D = 128
NUM_ITERS = 4
GAMMA = 0.1
EPS = 0.1

NC = 2
NS = 16
NW = NC * NS

C = 64
NBUF = 4
PH = 4
CH = NBUF * PH * (-(-E // (NW * C * NBUF * PH)))
CHP = CH // PH
EPT = CH * C
E_PAD = EPT * NW

RPT = 632
N_PAD = RPT * NS

_mesh = plsc.VectorSubcoreMesh(
    core_axis_name="c", subcore_axis_name="s", num_cores=NC, num_subcores=NS
)


@functools.partial(
    pl.kernel,
    out_type=jax.ShapeDtypeStruct((NC, N_PAD, D), jnp.float32),
    mesh=_mesh,
    scratch_types=[
        pltpu.VMEM((CHP, C), jnp.int32),
        pltpu.VMEM((CHP, C), jnp.int32),
        [pltpu.VMEM((C, D), jnp.float32) for _ in range(NBUF)],
        pltpu.VMEM_SHARED((N_PAD, D), jnp.float32),
        [pltpu.SemaphoreType.DMA for _ in range(NBUF)],
    ],
)
def _sc_propagate(h_hbm, src_hbm, dst_hbm, out_hbm, sidx, didx, rows, acc, sems):
    cid = lax.axis_index("c")
    sid = lax.axis_index("s")
    wid = sid * NC + cid

    def _zero(i, _):
        rows[0][i // 8, pl.ds((i % 8) * 16, 16)] = jnp.zeros((16,), jnp.float32)
        return 0

    lax.fori_loop(0, C * (D // 16), _zero, 0)

    base = sid * RPT
    spans = [(i * C, min(C, RPT - i * C)) for i in range((RPT + C - 1) // C)]
    for off, sz in spans:
        pltpu.sync_copy(rows[0].at[pl.ds(0, sz)], acc.at[pl.ds(base + off, sz)])
    plsc.subcore_barrier()

    for ph in range(PH):
        pltpu.sync_copy(src_hbm.at[wid, pl.ds(ph * CHP, CHP)], sidx)
        pltpu.sync_copy(dst_hbm.at[wid, pl.ds(ph * CHP, CHP)], didx)
        for b in range(NBUF):
            pltpu.async_copy(h_hbm.at[sidx.at[b]], rows[b], sems[b])

        def _round(j2, fire_next):
            for b in range(NBUF):
                c = j2 * NBUF + b
                pltpu.make_async_copy(h_hbm.at[sidx.at[b]], rows[b], sems[b]).wait()
                pltpu.sync_copy(rows[b], acc.at[didx.at[c]], add=True)
                if fire_next:
                    pltpu.async_copy(h_hbm.at[sidx.at[c + NBUF]], rows[b], sems[b])

        def _body(j2, _):
            _round(j2, True)
            return 0

        lax.fori_loop(0, CHP // NBUF - 1, _body, 0)
        _round(CHP // NBUF - 1, False)
    plsc.subcore_barrier()

    pltpu.sync_copy(acc.at[pl.ds(base, RPT)], out_hbm.at[cid, pl.ds(base, RPT)])


def _embed_body(x_ref, w_ref, b_ref, o_ref):
    z = lax.dot_general(
        x_ref[...], w_ref[...], (((1,), (1,)), ((), ())),
        preferred_element_type=jnp.float32,
    )
    o_ref[...] = jnp.maximum(z + b_ref[...], 0.0)


def _new_h(h_ref, p_ref, w_ref, lw_ref, b_ref):
    h = h_ref[...]
    p = p_ref[0] + p_ref[1]
    hwt = lax.dot_general(h, w_ref[...], (((1,), (1,)), ((), ())),
                          preferred_element_type=jnp.float32)
    hw = lax.dot_general(h, w_ref[...], (((1,), (0,)), ((), ())),
                         preferred_element_type=jnp.float32)
    plw = lax.dot_general(p, lw_ref[...], (((1,), (1,)), ((), ())),
                          preferred_element_type=jnp.float32)
    conv = hwt - hw - GAMMA * h + plw + b_ref[...]
    return h + EPS * jnp.tanh(conv)


def _update_body(h_ref, p_ref, w_ref, lw_ref, b_ref, o_ref):
    o_ref[...] = _new_h(h_ref, p_ref, w_ref, lw_ref, b_ref)


def _update_readout_body(h_ref, p_ref, w_ref, lw_ref, b_ref, rw_ref, rb_ref, o_ref):
    hn = _new_h(h_ref, p_ref, w_ref, lw_ref, b_ref)
    z = lax.dot_general(hn, rw_ref[...], (((1,), (1,)), ((), ())),
                        preferred_element_type=jnp.float32)
    o_ref[...] = z + rb_ref[...]


_ROWS_B = 1000
_GRID = (N // _ROWS_B,)
_row_spec = pl.BlockSpec((_ROWS_B, D), lambda i: (i, 0))
_parts_spec = pl.BlockSpec((NC, _ROWS_B, D), lambda i: (0, i, 0))
_mat_spec = pl.BlockSpec((D, D), lambda i: (0, 0))
_vec_spec = pl.BlockSpec((1, D), lambda i: (0, 0))
_out_struct = jax.ShapeDtypeStruct((N, D), jnp.float32)


def _tc_embed(x, w, b2):
    return pl.pallas_call(
        _embed_body, grid=_GRID,
        in_specs=[_row_spec, _mat_spec, _vec_spec],
        out_specs=_row_spec, out_shape=_out_struct,
    )(x, w, b2)


def _tc_update(h, parts, w, lw, b2):
    return pl.pallas_call(
        _update_body, grid=_GRID,
        in_specs=[_row_spec, _parts_spec, _mat_spec, _mat_spec, _vec_spec],
        out_specs=_row_spec, out_shape=_out_struct,
    )(h, parts, w, lw, b2)


def _tc_update_readout(h, parts, w, lw, b2, rw, rb2):
    return pl.pallas_call(
        _update_readout_body, grid=_GRID,
        in_specs=[_row_spec, _parts_spec, _mat_spec, _mat_spec, _vec_spec,
                  _mat_spec, _vec_spec],
        out_specs=_row_spec, out_shape=_out_struct,
    )(h, parts, w, lw, b2, rw, rb2)


def kernel(x, edge_index, emb_W, emb_b, W, bias, lin_W, readout_W, readout_b):
    ei = edge_index.astype(jnp.int32)
    pad = E_PAD - E
    pad_ar = np.arange(pad, dtype=np.int32)
    pad_src = jnp.asarray(pad_ar % N)
    pad_dst = jnp.asarray(N + (pad_ar % (N_PAD - N)))
    src = jnp.concatenate([ei[0], pad_src]).reshape(NW, CH, C)
    dst = jnp.concatenate([ei[1], pad_dst]).reshape(NW, CH, C)

    emb_b2 = emb_b.reshape(1, D)
    bias2 = bias.reshape(1, D)
    ro_b2 = readout_b.reshape(1, D)

    h = _tc_embed(x, emb_W, emb_b2)
    for _ in range(NUM_ITERS - 1):
        parts = _sc_propagate(h, src, dst)
        h = _tc_update(h, parts, W, lin_W, bias2)
    parts = _sc_propagate(h, src, dst)
    return _tc_update_readout(h, parts, W, lin_W, bias2, readout_W, ro_b2)
